# Initial kernel scaffold; baseline (speedup 1.0000x reference)
import jax
import jax.numpy as jnp
from jax.experimental import pallas as pl


def kernel(points_coor, points_fea, W, gamma, beta):
    raise NotImplementedError("write your pallas kernel here")


# speedup vs baseline: 8.5682x; 8.5682x over previous
import functools

import jax
import jax.numpy as jnp
from jax import lax
from jax.experimental import pallas as pl
from jax.experimental.pallas import tpu as pltpu
from jax.experimental.pallas import tpu_sc as plsc

_RADIUS = 0.2
_R2 = _RADIUS * _RADIUS
_G = 32
_B = 2
_N = 4096
_C = 256
_BN = _B * _N

_NC = 2
_NS = 16
_NW = _NC * _NS
_QW = _BN // _NW
_QC = 4
_NCHUNK = _QW // _QC


def _stage_a_body(fea_ref, xyz_ref, wf_ref, wc_ref, full_ref, f_ref):
    x3 = xyz_ref[...] * (1.0 / _RADIUS)
    wc = wc_ref[...]
    f = (x3[:, 0:1] * wc[0:1, :]
         + x3[:, 1:2] * wc[1:2, :]
         + x3[:, 2:3] * wc[2:3, :])
    full = jnp.dot(fea_ref[...], wf_ref[...],
                   preferred_element_type=jnp.float32) + f
    full_ref[...] = full
    f_ref[...] = f


def _stage_a(feaT, coorT, Wf, Wc):
    blk = 512
    grid = (_BN // blk,)
    return pl.pallas_call(
        _stage_a_body,
        grid=grid,
        in_specs=[
            pl.BlockSpec((blk, _C), lambda i: (i, 0)),
            pl.BlockSpec((blk, 3), lambda i: (i, 0)),
            pl.BlockSpec((_C, _C), lambda i: (0, 0)),
            pl.BlockSpec((3, _C), lambda i: (0, 0)),
        ],
        out_specs=[
            pl.BlockSpec((blk, _C), lambda i: (i, 0)),
            pl.BlockSpec((blk, _C), lambda i: (i, 0)),
        ],
        out_shape=[
            jax.ShapeDtypeStruct((_BN, _C), jnp.float32),
            jax.ShapeDtypeStruct((_BN, _C), jnp.float32),
        ],
    )(feaT, coorT, Wf, Wc)


_QB = 256


def _stage_b_body(qref, xref, idx_ref):
    b = pl.program_id(0)
    q = qref[0]
    x = xref[0]
    qsq = jnp.sum(q * q, axis=1, keepdims=True)
    xsq = jnp.sum(x * x, axis=0, keepdims=True)
    t = jnp.dot(q, x, preferred_element_type=jnp.float32)
    d = -2.0 * t + qsq + xsq
    inf = jnp.float32(jnp.inf)
    dwork = jnp.where(d <= _R2, d, inf)
    lane = lax.broadcasted_iota(jnp.int32, (_QB, _N), 1)
    b_off = b * _N
    cols = []
    i0 = None
    for g in range(_G):
        m = jnp.min(dwork, axis=1, keepdims=True)
        hit = dwork == m
        ii = jnp.min(jnp.where(hit, lane, _N), axis=1, keepdims=True)
        if g == 0:
            i0 = ii
            sel = ii
        else:
            sel = jnp.where(m <= _R2, ii, i0)
        cols.append(sel + b_off)
        dwork = jnp.where(lane == ii, inf, dwork)
    idx_ref[0] = jnp.concatenate(cols, axis=1)


def _stage_b(coor_bt, coor):
    grid = (_B, _N // _QB)
    return pl.pallas_call(
        _stage_b_body,
        grid=grid,
        in_specs=[
            pl.BlockSpec((1, _QB, 3), lambda b, i: (b, i, 0)),
            pl.BlockSpec((1, 3, _N), lambda b, i: (b, 0, 0)),
        ],
        out_specs=pl.BlockSpec((1, _QB, _G), lambda b, i: (b, i, 0)),
        out_shape=jax.ShapeDtypeStruct((_B, _N, _G), jnp.int32),
    )(coor_bt, coor)


def _stage_c_body(full_hbm, idx_hbm, sum_hbm, sq_hbm, max_hbm,
                  idx_v, rows_v, osum_v, osq_v, omax_v, sem):
    cid = lax.axis_index("c")
    sid = lax.axis_index("s")
    wid = sid * _NC + cid
    qbase = wid * _QW
    pltpu.sync_copy(idx_hbm.at[pl.ds(qbase * _G, _QW * _G)], idx_v)

    def chunk(c, carry):
        pltpu.async_copy(
            full_hbm.at[idx_v.at[pl.ds(c * (_QC * _G), _QC * _G)]],
            rows_v, sem).wait()
        for q in range(_QC):
            for j in range(_C // 16):
                def rbody(r8, acc):
                    s, ss, mx = acc
                    for u in range(8):
                        v = rows_v[q * _G + r8 * 8 + u, pl.ds(j * 16, 16)]
                        s = s + v
                        ss = ss + v * v
                        mx = jnp.maximum(mx, v)
                    return s, ss, mx
                z = jnp.zeros((16,), jnp.float32)
                ninf = jnp.full((16,), -jnp.inf, jnp.float32)
                s, ss, mx = lax.fori_loop(0, _G // 8, rbody, (z, z, ninf))
                osum_v[q, pl.ds(j * 16, 16)] = s
                osq_v[q, pl.ds(j * 16, 16)] = ss
                omax_v[q, pl.ds(j * 16, 16)] = mx
        row = qbase + c * _QC
        pltpu.sync_copy(osum_v, sum_hbm.at[pl.ds(row, _QC)])
        pltpu.sync_copy(osq_v, sq_hbm.at[pl.ds(row, _QC)])
        pltpu.sync_copy(omax_v, max_hbm.at[pl.ds(row, _QC)])
        return carry

    lax.fori_loop(0, _NCHUNK, chunk, 0)


def _stage_c(full, idx_flat):
    mesh = plsc.VectorSubcoreMesh(core_axis_name="c", subcore_axis_name="s")
    fn = functools.partial(
        pl.kernel,
        mesh=mesh,
        out_type=[
            jax.ShapeDtypeStruct((_BN, _C), jnp.float32),
            jax.ShapeDtypeStruct((_BN, _C), jnp.float32),
            jax.ShapeDtypeStruct((_BN, _C), jnp.float32),
        ],
        scratch_types=[
            pltpu.VMEM((_QW * _G,), jnp.int32),
            pltpu.VMEM((_QC * _G, _C), jnp.float32),
            pltpu.VMEM((_QC, _C), jnp.float32),
            pltpu.VMEM((_QC, _C), jnp.float32),
            pltpu.VMEM((_QC, _C), jnp.float32),
            pltpu.SemaphoreType.DMA,
        ],
    )(_stage_c_body)
    return fn(full, idx_flat)


_DBLK = 512


def _stage_d1_body(sum_ref, sq_ref, f_ref, s1_ref, s2_ref):
    i = pl.program_id(0)
    sv = sum_ref[...]
    qv = sq_ref[...]
    fv = f_ref[...]
    t1 = sv - jnp.float32(_G) * fv
    t2 = qv - 2.0 * fv * sv + jnp.float32(_G) * fv * fv
    p1 = jnp.sum(t1.reshape(_DBLK // 8, 8, _C), axis=0)
    p2 = jnp.sum(t2.reshape(_DBLK // 8, 8, _C), axis=0)

    @pl.when(i == 0)
    def _():
        s1_ref[...] = jnp.zeros_like(s1_ref)
        s2_ref[...] = jnp.zeros_like(s2_ref)

    s1_ref[...] += p1
    s2_ref[...] += p2


def _stage_d1(sumv, sqv, fproj):
    grid = (_BN // _DBLK,)
    return pl.pallas_call(
        _stage_d1_body,
        grid=grid,
        in_specs=[
            pl.BlockSpec((_DBLK, _C), lambda i: (i, 0)),
            pl.BlockSpec((_DBLK, _C), lambda i: (i, 0)),
            pl.BlockSpec((_DBLK, _C), lambda i: (i, 0)),
        ],
        out_specs=[
            pl.BlockSpec((8, _C), lambda i: (0, 0)),
            pl.BlockSpec((8, _C), lambda i: (0, 0)),
        ],
        out_shape=[
            jax.ShapeDtypeStruct((8, _C), jnp.float32),
            jax.ShapeDtypeStruct((8, _C), jnp.float32),
        ],
    )(sumv, sqv, fproj)


def _stage_d2_body(max_ref, f_ref, s1_ref, s2_ref, g_ref, b_ref, out_ref):
    m = jnp.float32(_B * _N * _G)
    s1 = jnp.sum(s1_ref[...], axis=0, keepdims=True)
    s2 = jnp.sum(s2_ref[...], axis=0, keepdims=True)
    mean = s1 / m
    var = s2 / m - mean * mean
    rstd = lax.rsqrt(var + 1e-5)
    a = g_ref[0:1, :] * rstd
    bb = b_ref[0:1, :] - mean * a
    y = (max_ref[...] - f_ref[...]) * a + bb
    out_ref[...] = jnp.maximum(y, 0.0)


def _stage_d2(maxv, fproj, s1, s2, gamma8, beta8):
    grid = (_BN // _DBLK,)
    return pl.pallas_call(
        _stage_d2_body,
        grid=grid,
        in_specs=[
            pl.BlockSpec((_DBLK, _C), lambda i: (i, 0)),
            pl.BlockSpec((_DBLK, _C), lambda i: (i, 0)),
            pl.BlockSpec((8, _C), lambda i: (0, 0)),
            pl.BlockSpec((8, _C), lambda i: (0, 0)),
            pl.BlockSpec((8, _C), lambda i: (0, 0)),
            pl.BlockSpec((8, _C), lambda i: (0, 0)),
        ],
        out_specs=pl.BlockSpec((_DBLK, _C), lambda i: (i, 0)),
        out_shape=jax.ShapeDtypeStruct((_BN, _C), jnp.float32),
    )(maxv, fproj, s1, s2, gamma8, beta8)


def kernel(points_coor, points_fea, W, gamma, beta):
    coor_bt = jnp.transpose(points_coor, (0, 2, 1))
    coorT = coor_bt.reshape(_BN, 3)
    feaT = jnp.transpose(points_fea, (0, 2, 1)).reshape(_BN, _C)
    Wf = jnp.transpose(W[:, :_C])
    Wc = jnp.transpose(W[:, _C:])

    full, fproj = _stage_a(feaT, coorT, Wf, Wc)
    idx = _stage_b(coor_bt, points_coor)
    sumv, sqv, maxv = _stage_c(full, idx.reshape(-1))
    s1, s2 = _stage_d1(sumv, sqv, fproj)
    gamma8 = jnp.broadcast_to(gamma.reshape(1, _C), (8, _C))
    beta8 = jnp.broadcast_to(beta.reshape(1, _C), (8, _C))
    out = _stage_d2(maxv, fproj, s1, s2, gamma8, beta8)
    return out.reshape(_B, _N, _C).transpose(0, 2, 1)


# speedup vs baseline: 9.0254x; 1.0534x over previous
import functools

import jax
import jax.numpy as jnp
from jax import lax
from jax.experimental import pallas as pl
from jax.experimental.pallas import tpu as pltpu
from jax.experimental.pallas import tpu_sc as plsc

_RADIUS = 0.2
_R2 = _RADIUS * _RADIUS
_G = 32
_B = 2
_N = 4096
_C = 256
_BN = _B * _N

_NC = 2
_NS = 16
_NW = _NC * _NS
_QW = _BN // _NW
_QC = 4
_NCHUNK = _QW // _QC


def _stage_a_body(fea_ref, xyz_ref, wf_ref, wc_ref, full_ref, f_ref):
    x3 = xyz_ref[...] * (1.0 / _RADIUS)
    wc = wc_ref[...]
    f = (x3[:, 0:1] * wc[0:1, :]
         + x3[:, 1:2] * wc[1:2, :]
         + x3[:, 2:3] * wc[2:3, :])
    full = jnp.dot(fea_ref[...], wf_ref[...],
                   preferred_element_type=jnp.float32) + f
    full_ref[...] = full
    f_ref[...] = f


def _stage_a(feaT, coorT, Wf, Wc):
    blk = 512
    grid = (_BN // blk,)
    return pl.pallas_call(
        _stage_a_body,
        grid=grid,
        in_specs=[
            pl.BlockSpec((blk, _C), lambda i: (i, 0)),
            pl.BlockSpec((blk, 3), lambda i: (i, 0)),
            pl.BlockSpec((_C, _C), lambda i: (0, 0)),
            pl.BlockSpec((3, _C), lambda i: (0, 0)),
        ],
        out_specs=[
            pl.BlockSpec((blk, _C), lambda i: (i, 0)),
            pl.BlockSpec((blk, _C), lambda i: (i, 0)),
        ],
        out_shape=[
            jax.ShapeDtypeStruct((_BN, _C), jnp.float32),
            jax.ShapeDtypeStruct((_BN, _C), jnp.float32),
        ],
    )(feaT, coorT, Wf, Wc)


_QB = 256


def _stage_b_body(qref, xref, idx_ref):
    b = pl.program_id(0)
    q = qref[0]
    x = xref[0]
    qsq = jnp.sum(q * q, axis=1, keepdims=True)
    xsq = jnp.sum(x * x, axis=0, keepdims=True)
    t = jnp.dot(q, x, preferred_element_type=jnp.float32)
    d = -2.0 * t + qsq + xsq
    inf = jnp.float32(jnp.inf)
    dwork = jnp.where(d <= _R2, d, inf)
    lane = lax.broadcasted_iota(jnp.int32, (_QB, _N), 1)
    b_off = b * _N
    cols = []
    i0 = None
    for g in range(_G):
        m = jnp.min(dwork, axis=1, keepdims=True)
        hit = dwork == m
        ii = jnp.min(jnp.where(hit, lane, _N), axis=1, keepdims=True)
        if g == 0:
            i0 = ii
            sel = ii
        else:
            sel = jnp.where(m <= _R2, ii, i0)
        cols.append(sel + b_off)
        dwork = jnp.where(lane == ii, inf, dwork)
    idx_ref[0] = jnp.concatenate(cols, axis=1)


def _stage_b(coor_bt, coor):
    grid = (_B, _N // _QB)
    return pl.pallas_call(
        _stage_b_body,
        grid=grid,
        in_specs=[
            pl.BlockSpec((1, _QB, 3), lambda b, i: (b, i, 0)),
            pl.BlockSpec((1, 3, _N), lambda b, i: (b, 0, 0)),
        ],
        out_specs=pl.BlockSpec((1, _QB, _G), lambda b, i: (b, i, 0)),
        out_shape=jax.ShapeDtypeStruct((_B, _N, _G), jnp.int32),
    )(coor_bt, coor)


def _stage_c_body(full_hbm, idx_hbm, sum_hbm, sq_hbm, max_hbm,
                  idx_v, rows0_v, rows1_v, osum_v, osq_v, omax_v,
                  sem0, sem1):
    cid = lax.axis_index("c")
    sid = lax.axis_index("s")
    wid = sid * _NC + cid
    qbase = wid * _QW
    pltpu.sync_copy(idx_hbm.at[pl.ds(qbase * _G, _QW * _G)], idx_v)

    def gather_src(c):
        return full_hbm.at[idx_v.at[pl.ds(c * (_QC * _G), _QC * _G)]]

    def reduce_chunk(rows_v, c):
        for q in range(_QC):
            for j in range(_C // 16):
                def rbody(r8, acc):
                    s, ss, mx = acc
                    for u in range(8):
                        v = rows_v[q * _G + r8 * 8 + u, pl.ds(j * 16, 16)]
                        s = s + v
                        ss = ss + v * v
                        mx = jnp.maximum(mx, v)
                    return s, ss, mx
                z = jnp.zeros((16,), jnp.float32)
                ninf = jnp.full((16,), -jnp.inf, jnp.float32)
                s, ss, mx = lax.fori_loop(0, _G // 8, rbody, (z, z, ninf))
                osum_v[q, pl.ds(j * 16, 16)] = s
                osq_v[q, pl.ds(j * 16, 16)] = ss
                omax_v[q, pl.ds(j * 16, 16)] = mx
        row = qbase + c * _QC
        pltpu.sync_copy(osum_v, sum_hbm.at[pl.ds(row, _QC)])
        pltpu.sync_copy(osq_v, sq_hbm.at[pl.ds(row, _QC)])
        pltpu.sync_copy(omax_v, max_hbm.at[pl.ds(row, _QC)])

    pltpu.async_copy(gather_src(0), rows0_v, sem0)

    def body(i, carry):
        c0 = 2 * i
        pltpu.make_async_copy(gather_src(c0), rows0_v, sem0).wait()
        pltpu.async_copy(gather_src(c0 + 1), rows1_v, sem1)
        reduce_chunk(rows0_v, c0)
        pltpu.make_async_copy(gather_src(c0 + 1), rows1_v, sem1).wait()

        @pl.when(i < _NCHUNK // 2 - 1)
        def _():
            pltpu.async_copy(gather_src(c0 + 2), rows0_v, sem0)

        reduce_chunk(rows1_v, c0 + 1)
        return carry

    lax.fori_loop(0, _NCHUNK // 2, body, 0)


def _stage_c(full, idx_flat):
    mesh = plsc.VectorSubcoreMesh(core_axis_name="c", subcore_axis_name="s")
    fn = functools.partial(
        pl.kernel,
        mesh=mesh,
        out_type=[
            jax.ShapeDtypeStruct((_BN, _C), jnp.float32),
            jax.ShapeDtypeStruct((_BN, _C), jnp.float32),
            jax.ShapeDtypeStruct((_BN, _C), jnp.float32),
        ],
        scratch_types=[
            pltpu.VMEM((_QW * _G,), jnp.int32),
            pltpu.VMEM((_QC * _G, _C), jnp.float32),
            pltpu.VMEM((_QC * _G, _C), jnp.float32),
            pltpu.VMEM((_QC, _C), jnp.float32),
            pltpu.VMEM((_QC, _C), jnp.float32),
            pltpu.VMEM((_QC, _C), jnp.float32),
            pltpu.SemaphoreType.DMA,
            pltpu.SemaphoreType.DMA,
        ],
    )(_stage_c_body)
    return fn(full, idx_flat)


_DBLK = 512


def _stage_d1_body(sum_ref, sq_ref, f_ref, s1_ref, s2_ref):
    i = pl.program_id(0)
    sv = sum_ref[...]
    qv = sq_ref[...]
    fv = f_ref[...]
    t1 = sv - jnp.float32(_G) * fv
    t2 = qv - 2.0 * fv * sv + jnp.float32(_G) * fv * fv
    p1 = jnp.sum(t1.reshape(_DBLK // 8, 8, _C), axis=0)
    p2 = jnp.sum(t2.reshape(_DBLK // 8, 8, _C), axis=0)

    @pl.when(i == 0)
    def _():
        s1_ref[...] = jnp.zeros_like(s1_ref)
        s2_ref[...] = jnp.zeros_like(s2_ref)

    s1_ref[...] += p1
    s2_ref[...] += p2


def _stage_d1(sumv, sqv, fproj):
    grid = (_BN // _DBLK,)
    return pl.pallas_call(
        _stage_d1_body,
        grid=grid,
        in_specs=[
            pl.BlockSpec((_DBLK, _C), lambda i: (i, 0)),
            pl.BlockSpec((_DBLK, _C), lambda i: (i, 0)),
            pl.BlockSpec((_DBLK, _C), lambda i: (i, 0)),
        ],
        out_specs=[
            pl.BlockSpec((8, _C), lambda i: (0, 0)),
            pl.BlockSpec((8, _C), lambda i: (0, 0)),
        ],
        out_shape=[
            jax.ShapeDtypeStruct((8, _C), jnp.float32),
            jax.ShapeDtypeStruct((8, _C), jnp.float32),
        ],
    )(sumv, sqv, fproj)


def _stage_d2_body(max_ref, f_ref, s1_ref, s2_ref, g_ref, b_ref, out_ref):
    m = jnp.float32(_B * _N * _G)
    s1 = jnp.sum(s1_ref[...], axis=0, keepdims=True)
    s2 = jnp.sum(s2_ref[...], axis=0, keepdims=True)
    mean = s1 / m
    var = s2 / m - mean * mean
    rstd = lax.rsqrt(var + 1e-5)
    a = g_ref[0:1, :] * rstd
    bb = b_ref[0:1, :] - mean * a
    y = (max_ref[...] - f_ref[...]) * a + bb
    out_ref[...] = jnp.maximum(y, 0.0)


def _stage_d2(maxv, fproj, s1, s2, gamma8, beta8):
    grid = (_BN // _DBLK,)
    return pl.pallas_call(
        _stage_d2_body,
        grid=grid,
        in_specs=[
            pl.BlockSpec((_DBLK, _C), lambda i: (i, 0)),
            pl.BlockSpec((_DBLK, _C), lambda i: (i, 0)),
            pl.BlockSpec((8, _C), lambda i: (0, 0)),
            pl.BlockSpec((8, _C), lambda i: (0, 0)),
            pl.BlockSpec((8, _C), lambda i: (0, 0)),
            pl.BlockSpec((8, _C), lambda i: (0, 0)),
        ],
        out_specs=pl.BlockSpec((_DBLK, _C), lambda i: (i, 0)),
        out_shape=jax.ShapeDtypeStruct((_BN, _C), jnp.float32),
    )(maxv, fproj, s1, s2, gamma8, beta8)


def kernel(points_coor, points_fea, W, gamma, beta):
    coor_bt = jnp.transpose(points_coor, (0, 2, 1))
    coorT = coor_bt.reshape(_BN, 3)
    feaT = jnp.transpose(points_fea, (0, 2, 1)).reshape(_BN, _C)
    Wf = jnp.transpose(W[:, :_C])
    Wc = jnp.transpose(W[:, _C:])

    full, fproj = _stage_a(feaT, coorT, Wf, Wc)
    idx = _stage_b(coor_bt, points_coor)
    sumv, sqv, maxv = _stage_c(full, idx.reshape(-1))
    s1, s2 = _stage_d1(sumv, sqv, fproj)
    gamma8 = jnp.broadcast_to(gamma.reshape(1, _C), (8, _C))
    beta8 = jnp.broadcast_to(beta.reshape(1, _C), (8, _C))
    out = _stage_d2(maxv, fproj, s1, s2, gamma8, beta8)
    return out.reshape(_B, _N, _C).transpose(0, 2, 1)


# speedup vs baseline: 9.7745x; 1.0830x over previous
import functools

import jax
import jax.numpy as jnp
from jax import lax
from jax.experimental import pallas as pl
from jax.experimental.pallas import tpu as pltpu
from jax.experimental.pallas import tpu_sc as plsc

_RADIUS = 0.2
_R2 = _RADIUS * _RADIUS
_G = 32
_B = 2
_N = 4096
_C = 256
_BN = _B * _N

_NC = 2
_NS = 16
_NW = _NC * _NS
_QW = _BN // _NW
_QC = 4
_NCHUNK = _QW // _QC


def _stage_a_body(fea_ref, xyz_ref, wf_ref, wc_ref, full_ref, f_ref):
    x3 = xyz_ref[...] * (1.0 / _RADIUS)
    wc = wc_ref[...]
    f = (x3[:, 0:1] * wc[0:1, :]
         + x3[:, 1:2] * wc[1:2, :]
         + x3[:, 2:3] * wc[2:3, :])
    full = jnp.dot(fea_ref[...], wf_ref[...],
                   preferred_element_type=jnp.float32) + f
    full_ref[...] = full
    f_ref[...] = f


def _stage_a(feaT, coorT, Wf, Wc):
    blk = 512
    grid = (_BN // blk,)
    return pl.pallas_call(
        _stage_a_body,
        grid=grid,
        in_specs=[
            pl.BlockSpec((blk, _C), lambda i: (i, 0)),
            pl.BlockSpec((blk, 3), lambda i: (i, 0)),
            pl.BlockSpec((_C, _C), lambda i: (0, 0)),
            pl.BlockSpec((3, _C), lambda i: (0, 0)),
        ],
        out_specs=[
            pl.BlockSpec((blk, _C), lambda i: (i, 0)),
            pl.BlockSpec((blk, _C), lambda i: (i, 0)),
        ],
        out_shape=[
            jax.ShapeDtypeStruct((_BN, _C), jnp.float32),
            jax.ShapeDtypeStruct((_BN, _C), jnp.float32),
        ],
    )(feaT, coorT, Wf, Wc)


_QB = 256


def _stage_b_body(qref, xref, idx_ref, *, b):
    q = qref[...]
    x = xref[...]
    qsq = jnp.sum(q * q, axis=1, keepdims=True)
    xsq = jnp.sum(x * x, axis=0, keepdims=True)
    t = jnp.dot(q, x, preferred_element_type=jnp.float32)
    d = -2.0 * t + qsq + xsq
    inf = jnp.float32(jnp.inf)
    dwork = jnp.where(d <= _R2, d, inf)
    lane = lax.broadcasted_iota(jnp.int32, (_QB, _N), 1)
    b_off = b * _N
    cols = []
    i0 = None
    for g in range(_G):
        m = jnp.min(dwork, axis=1, keepdims=True)
        hit = dwork == m
        ii = jnp.min(jnp.where(hit, lane, _N), axis=1, keepdims=True)
        if g == 0:
            i0 = ii
            sel = ii
        else:
            sel = jnp.where(m <= _R2, ii, i0)
        cols.append(sel + b_off)
        dwork = jnp.where(lane == ii, inf, dwork)
    idx_ref[...] = jnp.concatenate(cols, axis=1)


def _stage_b(coor_bt_b, coor_b, b):
    grid = (_N // _QB,)
    return pl.pallas_call(
        functools.partial(_stage_b_body, b=b),
        grid=grid,
        in_specs=[
            pl.BlockSpec((_QB, 3), lambda i: (i, 0)),
            pl.BlockSpec((3, _N), lambda i: (0, 0)),
        ],
        out_specs=pl.BlockSpec((_QB, _G), lambda i: (i, 0)),
        out_shape=jax.ShapeDtypeStruct((_N, _G), jnp.int32),
    )(coor_bt_b, coor_b)


def _stage_c_body(full_hbm, idx_hbm, sum_hbm, sq_hbm, max_hbm,
                  idx_v, rows0_v, rows1_v, osum_v, osq_v, omax_v,
                  sem0, sem1, *, qw, nchunk):
    cid = lax.axis_index("c")
    sid = lax.axis_index("s")
    wid = sid * _NC + cid
    qbase = wid * qw
    pltpu.sync_copy(idx_hbm.at[pl.ds(qbase * _G, qw * _G)], idx_v)

    def gather_src(c):
        return full_hbm.at[idx_v.at[pl.ds(c * (_QC * _G), _QC * _G)]]

    def reduce_chunk(rows_v, c):
        for q in range(_QC):
            for j in range(_C // 16):
                def rbody(r8, acc):
                    s, ss, mx = acc
                    for u in range(8):
                        v = rows_v[q * _G + r8 * 8 + u, pl.ds(j * 16, 16)]
                        s = s + v
                        ss = ss + v * v
                        mx = jnp.maximum(mx, v)
                    return s, ss, mx
                z = jnp.zeros((16,), jnp.float32)
                ninf = jnp.full((16,), -jnp.inf, jnp.float32)
                s, ss, mx = lax.fori_loop(0, _G // 8, rbody, (z, z, ninf))
                osum_v[q, pl.ds(j * 16, 16)] = s
                osq_v[q, pl.ds(j * 16, 16)] = ss
                omax_v[q, pl.ds(j * 16, 16)] = mx
        row = qbase + c * _QC
        pltpu.sync_copy(osum_v, sum_hbm.at[pl.ds(row, _QC)])
        pltpu.sync_copy(osq_v, sq_hbm.at[pl.ds(row, _QC)])
        pltpu.sync_copy(omax_v, max_hbm.at[pl.ds(row, _QC)])

    pltpu.async_copy(gather_src(0), rows0_v, sem0)

    def body(i, carry):
        c0 = 2 * i
        pltpu.make_async_copy(gather_src(c0), rows0_v, sem0).wait()
        pltpu.async_copy(gather_src(c0 + 1), rows1_v, sem1)
        reduce_chunk(rows0_v, c0)
        pltpu.make_async_copy(gather_src(c0 + 1), rows1_v, sem1).wait()

        @pl.when(i < nchunk // 2 - 1)
        def _():
            pltpu.async_copy(gather_src(c0 + 2), rows0_v, sem0)

        reduce_chunk(rows1_v, c0 + 1)
        return carry

    lax.fori_loop(0, nchunk // 2, body, 0)


def _stage_c(full, idx_flat, nq):
    qw = nq // _NW
    nchunk = qw // _QC
    mesh = plsc.VectorSubcoreMesh(core_axis_name="c", subcore_axis_name="s")
    fn = functools.partial(
        pl.kernel,
        mesh=mesh,
        out_type=[
            jax.ShapeDtypeStruct((nq, _C), jnp.float32),
            jax.ShapeDtypeStruct((nq, _C), jnp.float32),
            jax.ShapeDtypeStruct((nq, _C), jnp.float32),
        ],
        scratch_types=[
            pltpu.VMEM((qw * _G,), jnp.int32),
            pltpu.VMEM((_QC * _G, _C), jnp.float32),
            pltpu.VMEM((_QC * _G, _C), jnp.float32),
            pltpu.VMEM((_QC, _C), jnp.float32),
            pltpu.VMEM((_QC, _C), jnp.float32),
            pltpu.VMEM((_QC, _C), jnp.float32),
            pltpu.SemaphoreType.DMA,
            pltpu.SemaphoreType.DMA,
        ],
    )(functools.partial(_stage_c_body, qw=qw, nchunk=nchunk))
    return fn(full, idx_flat)


_DBLK = 512


def _stage_d1_body(sum_ref, sq_ref, f_ref, s1_ref, s2_ref):
    i = pl.program_id(0)
    sv = sum_ref[...]
    qv = sq_ref[...]
    fv = f_ref[...]
    t1 = sv - jnp.float32(_G) * fv
    t2 = qv - 2.0 * fv * sv + jnp.float32(_G) * fv * fv
    p1 = jnp.sum(t1.reshape(_DBLK // 8, 8, _C), axis=0)
    p2 = jnp.sum(t2.reshape(_DBLK // 8, 8, _C), axis=0)

    @pl.when(i == 0)
    def _():
        s1_ref[...] = jnp.zeros_like(s1_ref)
        s2_ref[...] = jnp.zeros_like(s2_ref)

    s1_ref[...] += p1
    s2_ref[...] += p2


def _stage_d1(sumv, sqv, fproj):
    grid = (_BN // _DBLK,)
    return pl.pallas_call(
        _stage_d1_body,
        grid=grid,
        in_specs=[
            pl.BlockSpec((_DBLK, _C), lambda i: (i, 0)),
            pl.BlockSpec((_DBLK, _C), lambda i: (i, 0)),
            pl.BlockSpec((_DBLK, _C), lambda i: (i, 0)),
        ],
        out_specs=[
            pl.BlockSpec((8, _C), lambda i: (0, 0)),
            pl.BlockSpec((8, _C), lambda i: (0, 0)),
        ],
        out_shape=[
            jax.ShapeDtypeStruct((8, _C), jnp.float32),
            jax.ShapeDtypeStruct((8, _C), jnp.float32),
        ],
    )(sumv, sqv, fproj)


def _stage_d2_body(max_ref, f_ref, s1_ref, s2_ref, g_ref, b_ref, out_ref):
    m = jnp.float32(_B * _N * _G)
    s1 = jnp.sum(s1_ref[...], axis=0, keepdims=True)
    s2 = jnp.sum(s2_ref[...], axis=0, keepdims=True)
    mean = s1 / m
    var = s2 / m - mean * mean
    rstd = lax.rsqrt(var + 1e-5)
    a = g_ref[0:1, :] * rstd
    bb = b_ref[0:1, :] - mean * a
    y = (max_ref[...] - f_ref[...]) * a + bb
    out_ref[...] = jnp.maximum(y, 0.0)


def _stage_d2(maxv, fproj, s1, s2, gamma8, beta8):
    grid = (_BN // _DBLK,)
    return pl.pallas_call(
        _stage_d2_body,
        grid=grid,
        in_specs=[
            pl.BlockSpec((_DBLK, _C), lambda i: (i, 0)),
            pl.BlockSpec((_DBLK, _C), lambda i: (i, 0)),
            pl.BlockSpec((8, _C), lambda i: (0, 0)),
            pl.BlockSpec((8, _C), lambda i: (0, 0)),
            pl.BlockSpec((8, _C), lambda i: (0, 0)),
            pl.BlockSpec((8, _C), lambda i: (0, 0)),
        ],
        out_specs=pl.BlockSpec((_DBLK, _C), lambda i: (i, 0)),
        out_shape=jax.ShapeDtypeStruct((_BN, _C), jnp.float32),
    )(maxv, fproj, s1, s2, gamma8, beta8)


def kernel(points_coor, points_fea, W, gamma, beta):
    coor_bt = jnp.transpose(points_coor, (0, 2, 1))
    coorT = coor_bt.reshape(_BN, 3)
    feaT = jnp.transpose(points_fea, (0, 2, 1)).reshape(_BN, _C)
    Wf = jnp.transpose(W[:, :_C])
    Wc = jnp.transpose(W[:, _C:])

    full, fproj = _stage_a(feaT, coorT, Wf, Wc)
    idx0 = _stage_b(coor_bt[0], points_coor[0], 0)
    sum0, sq0, max0 = _stage_c(full, idx0.reshape(-1), _N)
    idx1 = _stage_b(coor_bt[1], points_coor[1], 1)
    sum1, sq1, max1 = _stage_c(full, idx1.reshape(-1), _N)
    sumv = jnp.concatenate([sum0, sum1], axis=0)
    sqv = jnp.concatenate([sq0, sq1], axis=0)
    maxv = jnp.concatenate([max0, max1], axis=0)
    s1, s2 = _stage_d1(sumv, sqv, fproj)
    gamma8 = jnp.broadcast_to(gamma.reshape(1, _C), (8, _C))
    beta8 = jnp.broadcast_to(beta.reshape(1, _C), (8, _C))
    out = _stage_d2(maxv, fproj, s1, s2, gamma8, beta8)
    return out.reshape(_B, _N, _C).transpose(0, 2, 1)
